# baseline (speedup 1.0000x reference)
import functools

import jax
import jax.numpy as jnp
from jax import lax
from jax.experimental import pallas as pl
from jax.experimental.pallas import tpu as pltpu
from jax.experimental.pallas import tpu_sc as plsc

_N = 8 * 16 * 196
_C = 192
_D = 256
_K = 1024
_G = 8
_GS = _D // _G

_BM = 512
_NB = _N // _BM

_NC = 2
_NS = 16
_NW = _NC * _NS
_BPW = _N // _NW
_CHUNK = 112
_NCHUNK = _BPW // _CHUNK


def _encode_body(z_ref, w1_ref, b1_ref, w2_ref, b2_ref, gamma_ref,
                 beta_ref, cbt_ref, idx_ref):
    zb = z_ref[...]
    h = jnp.dot(zb, w1_ref[...], preferred_element_type=jnp.float32)
    h = jnp.maximum(h + b1_ref[...], 0.0)
    h = jnp.dot(h, w2_ref[...], preferred_element_type=jnp.float32)
    h = jnp.maximum(h + b2_ref[...], 0.0)
    parts = []
    for g in range(_G):
        seg = h[:, g * _GS:(g + 1) * _GS]
        m = jnp.mean(seg, axis=1, keepdims=True)
        d = seg - m
        v = jnp.mean(d * d, axis=1, keepdims=True)
        parts.append(d * lax.rsqrt(v + 1e-5))
    hn = jnp.concatenate(parts, axis=1)
    hq = hn * gamma_ref[...] + beta_ref[...]
    cbt = cbt_ref[...]
    scores = jnp.dot(hq, cbt, preferred_element_type=jnp.float32)
    scores = scores - 0.5 * jnp.sum(cbt * cbt, axis=0)
    smax = jnp.max(scores, axis=1, keepdims=True)
    lane = lax.broadcasted_iota(jnp.int32, scores.shape, 1)
    idx = jnp.min(jnp.where(scores == smax, lane, _K), axis=1)
    idx_ref[...] = idx.astype(jnp.int32).reshape(1, 1, _BM)


def _decode_body(zq_ref, wout_ref, bout_ref, out_ref):
    out_ref[...] = (
        jnp.dot(zq_ref[...], wout_ref[...], preferred_element_type=jnp.float32)
        + bout_ref[...]
    )


def _gather_body(table_hbm, idx_hbm, out_hbm, idx_v, rows_v, sem):
    wid = lax.axis_index("s") * _NC + lax.axis_index("c")
    base = wid * _BPW
    pltpu.sync_copy(idx_hbm.at[pl.ds(base, _BPW)], idx_v)
    for c in range(_NCHUNK):
        off = c * _CHUNK
        pltpu.async_copy(
            table_hbm.at[idx_v.at[pl.ds(off, _CHUNK)]], rows_v, sem
        ).wait()
        pltpu.sync_copy(rows_v, out_hbm.at[pl.ds(base + off, _CHUNK)])


def kernel(z, W1, b1, W2, b2, gamma, beta, codebook, W_out, b_out):
    z_flat = z.reshape(_N, _C)
    cbt = codebook.T

    idx3 = pl.pallas_call(
        _encode_body,
        grid=(_NB,),
        in_specs=[
            pl.BlockSpec((_BM, _C), lambda i: (i, 0)),
            pl.BlockSpec((_C, _D), lambda i: (0, 0)),
            pl.BlockSpec((1, _D), lambda i: (0, 0)),
            pl.BlockSpec((_D, _D), lambda i: (0, 0)),
            pl.BlockSpec((1, _D), lambda i: (0, 0)),
            pl.BlockSpec((1, _D), lambda i: (0, 0)),
            pl.BlockSpec((1, _D), lambda i: (0, 0)),
            pl.BlockSpec((_D, _K), lambda i: (0, 0)),
        ],
        out_specs=pl.BlockSpec((1, 1, _BM), lambda i: (i, 0, 0)),
        out_shape=jax.ShapeDtypeStruct((_NB, 1, _BM), jnp.int32),
    )(
        z_flat, W1, b1.reshape(1, _D), W2, b2.reshape(1, _D),
        gamma.reshape(1, _D), beta.reshape(1, _D), cbt,
    )
    idx = idx3.reshape(_N)

    mesh = plsc.VectorSubcoreMesh(
        core_axis_name="c", subcore_axis_name="s",
        num_cores=_NC, num_subcores=_NS,
    )
    gather = functools.partial(
        pl.kernel,
        out_type=jax.ShapeDtypeStruct((_N, _D), jnp.float32),
        mesh=mesh,
        scratch_types=[
            pltpu.VMEM((_BPW,), jnp.int32),
            pltpu.VMEM((_CHUNK, _D), jnp.float32),
            pltpu.SemaphoreType.DMA,
        ],
    )(_gather_body)
    zq_flat = gather(codebook, idx)

    out_flat = pl.pallas_call(
        _decode_body,
        grid=(_NB,),
        in_specs=[
            pl.BlockSpec((_BM, _D), lambda i: (i, 0)),
            pl.BlockSpec((_D, _C), lambda i: (0, 0)),
            pl.BlockSpec((1, _C), lambda i: (0, 0)),
        ],
        out_specs=pl.BlockSpec((_BM, _C), lambda i: (i, 0)),
        out_shape=jax.ShapeDtypeStruct((_N, _C), jnp.float32),
    )(zq_flat, W_out, b_out.reshape(1, _C))

    return out_flat.reshape(z.shape[:-1] + (_C,))


# speedup vs baseline: 1.1252x; 1.1252x over previous
import functools

import jax
import jax.numpy as jnp
import numpy as np
from jax import lax
from jax.experimental import pallas as pl
from jax.experimental.pallas import tpu as pltpu
from jax.experimental.pallas import tpu_sc as plsc

_N = 8 * 16 * 196
_F = 128
_HW = 196
_C = 192
_D = 256
_K = 1024
_G = 8
_GS = _D // _G

_BM = 784
_NB = _N // _BM

_FPB = 4
_DBM = _FPB * _HW
_NDB = _F // _FPB

_NC = 2
_NS = 16
_NW = _NC * _NS
_BPW = _N // _NW
_CHUNK = 112
_NCHUNK = _BPW // _CHUNK


def _encode_body(z_ref, w1_ref, b1_ref, w2_ref, b2_ref, gamma_ref,
                 beta_ref, cbt_ref, g_ref, gt_ref, idx_ref):
    zb = z_ref[...]
    h = jnp.dot(zb, w1_ref[...], preferred_element_type=jnp.float32)
    h = jnp.maximum(h + b1_ref[...], 0.0)
    h = jnp.dot(h, w2_ref[...], preferred_element_type=jnp.float32)
    h = jnp.maximum(h + b2_ref[...], 0.0)
    hp = lax.Precision.HIGHEST
    gmean = jnp.dot(h, g_ref[...], precision=hp,
                    preferred_element_type=jnp.float32)
    mean_full = jnp.dot(gmean, gt_ref[...], precision=hp,
                        preferred_element_type=jnp.float32)
    hc = h - mean_full
    gvar = jnp.dot(hc * hc, g_ref[...], precision=hp,
                   preferred_element_type=jnp.float32)
    var_full = jnp.dot(gvar, gt_ref[...], precision=hp,
                       preferred_element_type=jnp.float32)
    hq = hc * lax.rsqrt(var_full + 1e-5) * gamma_ref[...] + beta_ref[...]
    cbt = cbt_ref[...]
    scores = jnp.dot(hq, cbt, preferred_element_type=jnp.float32)
    scores = scores - 0.5 * jnp.sum(cbt * cbt, axis=0)
    smax = jnp.max(scores, axis=1, keepdims=True)
    lane = lax.broadcasted_iota(jnp.int32, scores.shape, 1)
    idx = jnp.min(jnp.where(scores == smax, lane, _K), axis=1)
    idx_ref[...] = idx.astype(jnp.int32).reshape(1, 1, _BM)


def _decode_body(zq_ref, wout_ref, bout_ref, out_ref):
    out_ref[...] = (
        jnp.dot(zq_ref[...], wout_ref[...], preferred_element_type=jnp.float32)
        + bout_ref[...]
    )


def _gather_body(table_hbm, idx_hbm, out_hbm, idx_v, rows_v, sem):
    wid = lax.axis_index("s") * _NC + lax.axis_index("c")
    base = wid * _BPW
    pltpu.sync_copy(idx_hbm.at[wid, 0], idx_v)
    for c in range(_NCHUNK):
        off = c * _CHUNK
        pltpu.async_copy(
            table_hbm.at[idx_v.at[pl.ds(off, _CHUNK)]], rows_v, sem
        ).wait()
        pltpu.sync_copy(rows_v, out_hbm.at[pl.ds(base + off, _CHUNK)])


def kernel(z, W1, b1, W2, b2, gamma, beta, codebook, W_out, b_out):
    z_flat = z.reshape(_N, _C)
    cbt = codebook.T
    g_ind = jnp.asarray(
        np.kron(np.eye(_G, dtype=np.float32), np.ones((_GS, 1), np.float32))
        / _GS
    )
    gt_ind = jnp.asarray(
        np.kron(np.eye(_G, dtype=np.float32), np.ones((1, _GS), np.float32))
    )

    idx = pl.pallas_call(
        _encode_body,
        grid=(_NB,),
        in_specs=[
            pl.BlockSpec((_BM, _C), lambda i: (i, 0)),
            pl.BlockSpec((_C, _D), lambda i: (0, 0)),
            pl.BlockSpec((1, _D), lambda i: (0, 0)),
            pl.BlockSpec((_D, _D), lambda i: (0, 0)),
            pl.BlockSpec((1, _D), lambda i: (0, 0)),
            pl.BlockSpec((1, _D), lambda i: (0, 0)),
            pl.BlockSpec((1, _D), lambda i: (0, 0)),
            pl.BlockSpec((_D, _K), lambda i: (0, 0)),
            pl.BlockSpec((_D, _G), lambda i: (0, 0)),
            pl.BlockSpec((_G, _D), lambda i: (0, 0)),
        ],
        out_specs=pl.BlockSpec((1, 1, _BM), lambda i: (i, 0, 0)),
        out_shape=jax.ShapeDtypeStruct((_NB, 1, _BM), jnp.int32),
    )(
        z_flat, W1, b1.reshape(1, _D), W2, b2.reshape(1, _D),
        gamma.reshape(1, _D), beta.reshape(1, _D), cbt, g_ind, gt_ind,
    )

    mesh = plsc.VectorSubcoreMesh(
        core_axis_name="c", subcore_axis_name="s",
        num_cores=_NC, num_subcores=_NS,
    )
    gather = functools.partial(
        pl.kernel,
        out_type=jax.ShapeDtypeStruct((_N, _D), jnp.float32),
        mesh=mesh,
        scratch_types=[
            pltpu.VMEM((_BPW,), jnp.int32),
            pltpu.VMEM((_CHUNK, _D), jnp.float32),
            pltpu.SemaphoreType.DMA,
        ],
    )(_gather_body)
    zq_flat = gather(codebook, idx)

    out3 = pl.pallas_call(
        _decode_body,
        grid=(_NDB,),
        in_specs=[
            pl.BlockSpec((_DBM, _D), lambda i: (i, 0)),
            pl.BlockSpec((_D, _C), lambda i: (0, 0)),
            pl.BlockSpec((1, _C), lambda i: (0, 0)),
        ],
        out_specs=pl.BlockSpec((_DBM, _C), lambda i: (i, 0)),
        out_shape=jax.ShapeDtypeStruct((_N, _C), jnp.float32),
    )(zq_flat, W_out, b_out.reshape(1, _C))

    return out3.reshape(z.shape[:-1] + (_C,))


# speedup vs baseline: 2.6814x; 2.3830x over previous
import functools

import jax
import jax.numpy as jnp
import numpy as np
from jax import lax
from jax.experimental import pallas as pl
from jax.experimental.pallas import tpu as pltpu
from jax.experimental.pallas import tpu_sc as plsc

_N = 8 * 16 * 196
_F = 128
_HW = 196
_C = 192
_D = 256
_K = 1024
_G = 8

_FPB = 4
_NB = _F // _FPB
_BM = _FPB * _HW

_NC = 2
_NS = 16
_NW = _NC * _NS
_BPW = _N // _NW
_CHUNK = 112
_NCHUNK = _BPW // _CHUNK


def _encode_body(z_ref, w1t_ref, b1_ref, w2t_ref, b2_ref, gamma_ref,
                 beta_ref, cb_ref, idx_ref):
    gs = _D // _G
    cb = cb_ref[...]
    cbn = jnp.sum(cb * cb, axis=1, keepdims=True)
    idx_parts = []
    for f in range(_FPB):
        zf = z_ref[f * _C:(f + 1) * _C, :]
        h = jnp.dot(w1t_ref[...], zf, preferred_element_type=jnp.float32)
        h = jnp.maximum(h + b1_ref[...], 0.0)
        h = jnp.dot(w2t_ref[...], h, preferred_element_type=jnp.float32)
        h = jnp.maximum(h + b2_ref[...], 0.0)
        parts = []
        for g in range(_G):
            seg = h[g * gs:(g + 1) * gs, :]
            m = jnp.mean(seg, axis=0, keepdims=True)
            d = seg - m
            v = jnp.mean(d * d, axis=0, keepdims=True)
            parts.append(d * lax.rsqrt(v + 1e-5))
        hc = jnp.concatenate(parts, axis=0)
        hq = hc * gamma_ref[...] + beta_ref[...]
        hqn = jnp.sum(hq * hq, axis=0, keepdims=True)
        m = jnp.dot(cb, hq, preferred_element_type=jnp.float32)
        d2 = (hqn - 2.0 * m) + cbn
        dmin = jnp.min(d2, axis=0, keepdims=True)
        row = lax.broadcasted_iota(jnp.int32, d2.shape, 0)
        idx_parts.append(jnp.min(jnp.where(d2 == dmin, row, _K), axis=0))
    idx = jnp.concatenate(idx_parts)
    idx_ref[...] = idx.astype(jnp.int32).reshape(1, 1, _BM)


def _decode_body(zq_ref, wout_ref, bout_ref, out_ref):
    y = (
        jnp.dot(zq_ref[...], wout_ref[...], preferred_element_type=jnp.float32)
        + bout_ref[...]
    )
    yt = y.T
    for f in range(_FPB):
        out_ref[f] = yt[:, f * _HW:(f + 1) * _HW]


def _gather_body(table_hbm, idx_hbm, out_hbm, idx_v, rows_v, sem):
    wid = lax.axis_index("s") * _NC + lax.axis_index("c")
    base = wid * _BPW
    pltpu.sync_copy(idx_hbm.at[wid, 0], idx_v)
    for c in range(_NCHUNK):
        off = c * _CHUNK
        pltpu.async_copy(
            table_hbm.at[idx_v.at[pl.ds(off, _CHUNK)]], rows_v, sem
        ).wait()
        pltpu.sync_copy(rows_v, out_hbm.at[pl.ds(base + off, _CHUNK)])


def kernel(z, W1, b1, W2, b2, gamma, beta, codebook, W_out, b_out):
    B, T = z.shape[0], z.shape[1]
    zt = jnp.swapaxes(z, 2, 3).reshape(_F * _C, _HW)

    idx3 = pl.pallas_call(
        _encode_body,
        grid=(_NB,),
        in_specs=[
            pl.BlockSpec((_FPB * _C, _HW), lambda i: (i, 0)),
            pl.BlockSpec((_D, _C), lambda i: (0, 0)),
            pl.BlockSpec((_D, 1), lambda i: (0, 0)),
            pl.BlockSpec((_D, _D), lambda i: (0, 0)),
            pl.BlockSpec((_D, 1), lambda i: (0, 0)),
            pl.BlockSpec((_D, 1), lambda i: (0, 0)),
            pl.BlockSpec((_D, 1), lambda i: (0, 0)),
            pl.BlockSpec((_K, _D), lambda i: (0, 0)),
        ],
        out_specs=pl.BlockSpec((1, 1, _BM), lambda i: (i, 0, 0)),
        out_shape=jax.ShapeDtypeStruct((_NB, 1, _BM), jnp.int32),
    )(
        zt, W1.T, b1.reshape(_D, 1), W2.T, b2.reshape(_D, 1),
        gamma.reshape(_D, 1), beta.reshape(_D, 1), codebook,
    )

    mesh = plsc.VectorSubcoreMesh(
        core_axis_name="c", subcore_axis_name="s",
        num_cores=_NC, num_subcores=_NS,
    )
    gather = functools.partial(
        pl.kernel,
        out_type=jax.ShapeDtypeStruct((_N, _D), jnp.float32),
        mesh=mesh,
        scratch_types=[
            pltpu.VMEM((_BPW,), jnp.int32),
            pltpu.VMEM((_CHUNK, _D), jnp.float32),
            pltpu.SemaphoreType.DMA,
        ],
    )(_gather_body)
    zq_flat = gather(codebook, idx3)

    out_t = pl.pallas_call(
        _decode_body,
        grid=(_NB,),
        in_specs=[
            pl.BlockSpec((_BM, _D), lambda i: (i, 0)),
            pl.BlockSpec((_D, _C), lambda i: (0, 0)),
            pl.BlockSpec((1, _C), lambda i: (0, 0)),
        ],
        out_specs=pl.BlockSpec((_FPB, _C, _HW), lambda i: (i, 0, 0)),
        out_shape=jax.ShapeDtypeStruct((_F, _C, _HW), jnp.float32),
    )(zq_flat, W_out, b_out.reshape(1, _C))

    return jnp.swapaxes(out_t.reshape(B, T, _C, _HW), 2, 3)


# speedup vs baseline: 2.9427x; 1.0975x over previous
import functools

import jax
import jax.numpy as jnp
import numpy as np
from jax import lax
from jax.experimental import pallas as pl
from jax.experimental.pallas import tpu as pltpu
from jax.experimental.pallas import tpu_sc as plsc

_N = 8 * 16 * 196
_F = 128
_HW = 196
_C = 192
_D = 256
_K = 1024
_G = 8

_FPB = 8
_NB = _F // _FPB
_BM = _FPB * _HW

_NC = 2
_NS = 16
_NW = _NC * _NS
_BPW = _N // _NW
_CHUNK = 112
_NCHUNK = _BPW // _CHUNK


def _encode_body(z_ref, w1t_ref, b1_ref, w2t_ref, b2_ref, gamma_ref,
                 beta_ref, cb_ref, idx_ref):
    gs = _D // _G
    cb = cb_ref[...]
    cbn = jnp.sum(cb * cb, axis=1, keepdims=True)
    idx_parts = []
    for f in range(_FPB):
        zf = z_ref[f * _C:(f + 1) * _C, :]
        h = jnp.dot(w1t_ref[...], zf, preferred_element_type=jnp.float32)
        h = jnp.maximum(h + b1_ref[...], 0.0)
        h = jnp.dot(w2t_ref[...], h, preferred_element_type=jnp.float32)
        h = jnp.maximum(h + b2_ref[...], 0.0)
        parts = []
        for g in range(_G):
            seg = h[g * gs:(g + 1) * gs, :]
            m = jnp.mean(seg, axis=0, keepdims=True)
            d = seg - m
            v = jnp.mean(d * d, axis=0, keepdims=True)
            parts.append(d * lax.rsqrt(v + 1e-5))
        hc = jnp.concatenate(parts, axis=0)
        hq = hc * gamma_ref[...] + beta_ref[...]
        hqn = jnp.sum(hq * hq, axis=0, keepdims=True)
        m = jnp.dot(cb, hq, preferred_element_type=jnp.float32)
        d2 = (hqn - 2.0 * m) + cbn
        dmin = jnp.min(d2, axis=0, keepdims=True)
        row = lax.broadcasted_iota(jnp.int32, d2.shape, 0)
        idx_parts.append(jnp.min(jnp.where(d2 == dmin, row, _K), axis=0))
    idx = jnp.concatenate(idx_parts)
    idx_ref[...] = idx.astype(jnp.int32).reshape(1, 1, _BM)


def _decode_body(zq_ref, wout_ref, bout_ref, out_ref):
    yt = lax.dot_general(
        wout_ref[...], zq_ref[...], (((0,), (1,)), ((), ())),
        preferred_element_type=jnp.float32,
    ) + bout_ref[...]
    for f in range(_FPB):
        out_ref[f] = yt[:, f * _HW:(f + 1) * _HW]


def _gather_body(table_hbm, idx_hbm, out_hbm, idx_v, rows_v, sem):
    wid = lax.axis_index("s") * _NC + lax.axis_index("c")
    base = wid * _BPW
    wpr = _BM // _BPW
    pltpu.sync_copy(idx_hbm.at[wid // wpr, 0], idx_v)
    voff = (wid % wpr) * _BPW
    for c in range(_NCHUNK):
        off = c * _CHUNK
        pltpu.async_copy(
            table_hbm.at[idx_v.at[pl.ds(voff + off, _CHUNK)]], rows_v, sem
        ).wait()
        pltpu.sync_copy(rows_v, out_hbm.at[pl.ds(base + off, _CHUNK)])


def kernel(z, W1, b1, W2, b2, gamma, beta, codebook, W_out, b_out):
    B, T = z.shape[0], z.shape[1]
    zt = jnp.swapaxes(z, 2, 3).reshape(_F * _C, _HW)

    idx3 = pl.pallas_call(
        _encode_body,
        grid=(_NB,),
        in_specs=[
            pl.BlockSpec((_FPB * _C, _HW), lambda i: (i, 0)),
            pl.BlockSpec((_D, _C), lambda i: (0, 0)),
            pl.BlockSpec((_D, 1), lambda i: (0, 0)),
            pl.BlockSpec((_D, _D), lambda i: (0, 0)),
            pl.BlockSpec((_D, 1), lambda i: (0, 0)),
            pl.BlockSpec((_D, 1), lambda i: (0, 0)),
            pl.BlockSpec((_D, 1), lambda i: (0, 0)),
            pl.BlockSpec((_K, _D), lambda i: (0, 0)),
        ],
        out_specs=pl.BlockSpec((1, 1, _BM), lambda i: (i, 0, 0)),
        out_shape=jax.ShapeDtypeStruct((_NB, 1, _BM), jnp.int32),
    )(
        zt, W1.T, b1.reshape(_D, 1), W2.T, b2.reshape(_D, 1),
        gamma.reshape(_D, 1), beta.reshape(_D, 1), codebook,
    )

    mesh = plsc.VectorSubcoreMesh(
        core_axis_name="c", subcore_axis_name="s",
        num_cores=_NC, num_subcores=_NS,
    )
    gather = functools.partial(
        pl.kernel,
        out_type=jax.ShapeDtypeStruct((_N, _D), jnp.float32),
        mesh=mesh,
        scratch_types=[
            pltpu.VMEM((_BM,), jnp.int32),
            pltpu.VMEM((_CHUNK, _D), jnp.float32),
            pltpu.SemaphoreType.DMA,
        ],
    )(_gather_body)
    zq_flat = gather(codebook, idx3)

    out_t = pl.pallas_call(
        _decode_body,
        grid=(_NB,),
        in_specs=[
            pl.BlockSpec((_BM, _D), lambda i: (i, 0)),
            pl.BlockSpec((_D, _C), lambda i: (0, 0)),
            pl.BlockSpec((_C, 1), lambda i: (0, 0)),
        ],
        out_specs=pl.BlockSpec((_FPB, _C, _HW), lambda i: (i, 0, 0)),
        out_shape=jax.ShapeDtypeStruct((_F, _C, _HW), jnp.float32),
    )(zq_flat, W_out, b_out.reshape(_C, 1))

    return jnp.swapaxes(out_t.reshape(B, T, _C, _HW), 2, 3)
